# baseline (speedup 1.0000x reference)
import functools

import jax
import jax.numpy as jnp
from jax import lax
from jax.experimental import pallas as pl
from jax.experimental.pallas import tpu as pltpu
from jax.experimental.pallas import tpu_sc as plsc

BATCH = 16384
EMB = 64
HID = 256

_NC = 2
_NS = 16
_NW = _NC * _NS
_ROWS_PER_W = BATCH // _NW
_G = 16
_NGRP = _ROWS_PER_W // _G
_N_STREAM = 368


def _gather_body(uid_hbm, iid_hbm, uemb_hbm, iemb_hbm, uout_hbm, iout_hbm,
                 idx_u, idx_i, rows_u, rows_i, sem_s, sem_d):
    wid = lax.axis_index("s") * _NC + lax.axis_index("c")
    base = wid * _ROWS_PER_W
    pltpu.sync_copy(uid_hbm.at[pl.ds(base, _ROWS_PER_W)], idx_u)
    pltpu.sync_copy(iid_hbm.at[pl.ds(base, _ROWS_PER_W)], idx_i)

    for idx_ref, emb_hbm, out_hbm, rows in (
            (idx_u, uemb_hbm, uout_hbm, rows_u),
            (idx_i, iemb_hbm, iout_hbm, rows_i)):
        def issue_s(g, carry):
            vec = idx_ref[pl.ds(g * _G, _G)]
            for k in range(_G):
                r = vec[k]
                i = g * _G + k
                pltpu.async_copy(emb_hbm.at[pl.ds(r, 1)],
                                 rows.at[pl.ds(i, 1)], sem_s)
            return carry

        def issue_d(g, carry):
            vec = idx_ref[pl.ds(g * _G, _G)]
            for k in range(_G):
                r = vec[k]
                i = g * _G + k
                pltpu.async_copy(emb_hbm.at[pl.ds(r, 1)],
                                 out_hbm.at[pl.ds(base + i, 1)], sem_d)
            return carry

        lax.fori_loop(0, _N_STREAM // _G, issue_s, 0)
        lax.fori_loop(_N_STREAM // _G, _NGRP, issue_d, 0)

    for emb_hbm, out_hbm, rows in ((uemb_hbm, uout_hbm, rows_u),
                                   (iemb_hbm, iout_hbm, rows_i)):
        pltpu.make_async_copy(emb_hbm.at[pl.ds(0, _N_STREAM)], rows,
                              sem_s).wait()
        pltpu.sync_copy(rows, out_hbm.at[pl.ds(base, _N_STREAM)])
    for emb_hbm, out_hbm in ((uemb_hbm, uout_hbm), (iemb_hbm, iout_hbm)):
        pltpu.make_async_copy(
            emb_hbm.at[pl.ds(0, _ROWS_PER_W - _N_STREAM)],
            out_hbm.at[pl.ds(base + _N_STREAM, _ROWS_PER_W - _N_STREAM)],
            sem_d).wait()


def _sc_gather(user_id, item_id, uemb3d, iemb3d):
    mesh = plsc.VectorSubcoreMesh(core_axis_name="c", subcore_axis_name="s")
    out_type = (
        jax.ShapeDtypeStruct((BATCH, EMB), jnp.float32),
        jax.ShapeDtypeStruct((BATCH, EMB), jnp.float32),
    )
    scratch = [
        pltpu.VMEM((_ROWS_PER_W,), jnp.int32),
        pltpu.VMEM((_ROWS_PER_W,), jnp.int32),
        pltpu.VMEM((_N_STREAM, EMB), jnp.float32),
        pltpu.VMEM((_N_STREAM, EMB), jnp.float32),
        pltpu.SemaphoreType.DMA,
        pltpu.SemaphoreType.DMA,
    ]
    return pl.kernel(
        _gather_body, mesh=mesh, out_type=out_type, scratch_types=scratch,
        name="ncf_sc_gather",
    )(user_id, item_id, uemb3d, iemb3d)


_BLK = 2048


def _mlp_body(u_ref, i_ref, w1u_ref, w1i_ref, b1_ref, w2_ref, b2_ref, o_ref):
    h = (jnp.dot(u_ref[...], w1u_ref[...], preferred_element_type=jnp.float32)
         + jnp.dot(i_ref[...], w1i_ref[...], preferred_element_type=jnp.float32)
         + b1_ref[...])
    h = jnp.maximum(h, 0.0)
    s = jnp.sum(h * w2_ref[...], axis=1, keepdims=True) + b2_ref[...]
    o_ref[...] = 1.0 / (1.0 + jnp.exp(-s))


def _tc_mlp(uvec, ivec, w1u, w1i, b1r, w2r, b2r):
    grid = (BATCH // _BLK,)
    return pl.pallas_call(
        _mlp_body,
        grid=grid,
        in_specs=[
            pl.BlockSpec((_BLK, EMB), lambda i: (i, 0)),
            pl.BlockSpec((_BLK, EMB), lambda i: (i, 0)),
            pl.BlockSpec((EMB, HID), lambda i: (0, 0)),
            pl.BlockSpec((EMB, HID), lambda i: (0, 0)),
            pl.BlockSpec((1, HID), lambda i: (0, 0)),
            pl.BlockSpec((1, HID), lambda i: (0, 0)),
            pl.BlockSpec((1, 1), lambda i: (0, 0)),
        ],
        out_specs=pl.BlockSpec((_BLK, 1), lambda i: (i, 0)),
        out_shape=jax.ShapeDtypeStruct((BATCH, 1), jnp.float32),
    )(uvec, ivec, w1u, w1i, b1r, w2r, b2r)


def kernel(user_id, item_id, user_emb, item_emb, W1, b1, W2, b2):
    uvec, ivec = _sc_gather(user_id.astype(jnp.int32), item_id.astype(jnp.int32),
                            user_emb, item_emb)
    w1u = W1[:EMB]
    w1i = W1[EMB:]
    b1r = b1.reshape(1, HID)
    w2r = W2.reshape(1, HID)
    b2r = b2.reshape(1, 1)
    return _tc_mlp(uvec, ivec, w1u, w1i, b1r, w2r, b2r)
